# baseline (speedup 1.0000x reference)
import dataclasses
import functools

import jax
import jax.numpy as jnp
from jax import lax
from jax.experimental import pallas as pl
from jax.experimental.pallas import tpu as pltpu
from jax.experimental.pallas import tpu_sc as plsc

N_USERS = 3000
N_NODES = 10000
D = 128
N_EDGES = 320000
HALF = 5000

NC = 2
NS = 16
NW = NC * NS
CHUNK = N_EDGES // NW
CAP = 10240
TRASH = 5000
RPT = 320
UROWS = NS * RPT
BLK = 128
SUB = 40

_mesh = plsc.VectorSubcoreMesh(core_axis_name="core", subcore_axis_name="subcore")

_cparams = pltpu.CompilerParams()
if "needs_layout_passes" in pltpu.CompilerParams.__dataclass_fields__:
    _cparams = dataclasses.replace(_cparams, needs_layout_passes=False)

f32 = jnp.float32
i32 = jnp.int32


def _rsqrt16(x):
    i = plsc.bitcast(x, i32)
    i = jnp.int32(0x5F3759DF) - (i >> 1)
    y = plsc.bitcast(i, f32)
    for _ in range(4):
        y = y * (1.5 - 0.5 * x * y * y)
    return y


@jax.jit
def _p1(src, dst):
    @functools.partial(
        pl.kernel,
        out_type=(
            jax.ShapeDtypeStruct((NC * NW * CAP,), i32),
            jax.ShapeDtypeStruct((NC * NW * CAP,), i32),
            jax.ShapeDtypeStruct((NC * NW * 16,), i32),
        ),
        mesh=_mesh,
        compiler_params=_cparams,
        scratch_types=[
            pltpu.VMEM((CHUNK,), i32),
            pltpu.VMEM((CHUNK,), i32),
            pltpu.VMEM((CAP,), i32),
            pltpu.VMEM((CAP,), i32),
            pltpu.VMEM((CAP,), i32),
            pltpu.VMEM((CAP,), i32),
            pltpu.VMEM((16,), i32),
        ],
    )
    def k(src_h, dst_h, sl_h, dl_h, cnt_h, sbuf, dbuf, l0s, l0d, l1s, l1d, cbuf):
        c = lax.axis_index("core")
        t = lax.axis_index("subcore")
        w = c * NS + t
        pltpu.sync_copy(src_h.at[pl.ds(w * CHUNK, CHUNK)], sbuf)
        pltpu.sync_copy(dst_h.at[pl.ds(w * CHUNK, CHUNK)], dbuf)

        def step(i, ns):
            n0, n1 = ns
            sv = sbuf[pl.ds(i * 16, 16)]
            dv = dbuf[pl.ds(i * 16, 16)]
            m0 = dv < HALF
            m1 = jnp.logical_not(m0)
            plsc.store_compressed(l0s.at[pl.ds(n0, 16)], sv, mask=m0)
            plsc.store_compressed(l0d.at[pl.ds(n0, 16)], dv, mask=m0)
            plsc.store_compressed(l1s.at[pl.ds(n1, 16)], sv, mask=m1)
            plsc.store_compressed(l1d.at[pl.ds(n1, 16)], dv - HALF, mask=m1)
            n0 = n0 + plsc.all_reduce_population_count(m0)[0]
            n1 = n1 + plsc.all_reduce_population_count(m1)[0]
            return n0, n1

        n0, n1 = lax.fori_loop(0, CHUNK // 16, step, (i32(0), i32(0)))

        zeros16 = jnp.zeros((16,), i32)
        trash16 = jnp.full((16,), TRASH, i32)
        for j in range(BLK // 16):
            l0s[pl.ds(n0 + j * 16, 16)] = zeros16
            l0d[pl.ds(n0 + j * 16, 16)] = trash16
            l1s[pl.ds(n1 + j * 16, 16)] = zeros16
            l1d[pl.ds(n1 + j * 16, 16)] = trash16

        pltpu.sync_copy(l0s, sl_h.at[pl.ds(w * CAP, CAP)])
        pltpu.sync_copy(l0d, dl_h.at[pl.ds(w * CAP, CAP)])
        pltpu.sync_copy(l1s, sl_h.at[pl.ds((NW + w) * CAP, CAP)])
        pltpu.sync_copy(l1d, dl_h.at[pl.ds((NW + w) * CAP, CAP)])
        cbuf[pl.ds(0, 16)] = jnp.full((16,), 1, i32) * n0
        pltpu.sync_copy(cbuf, cnt_h.at[pl.ds(w * 16, 16)])
        cbuf[pl.ds(0, 16)] = jnp.full((16,), 1, i32) * n1
        pltpu.sync_copy(cbuf, cnt_h.at[pl.ds((NW + w) * 16, 16)])

    return k(src, dst)


@jax.jit
def _p2(dl, cnt, x):
    @functools.partial(
        pl.kernel,
        out_type=(
            jax.ShapeDtypeStruct((NC * UROWS,), f32),
            jax.ShapeDtypeStruct((N_NODES, D), f32),
        ),
        mesh=_mesh,
        compiler_params=_cparams,
        scratch_types=[
            pltpu.VMEM_SHARED((UROWS,), f32),
            pltpu.VMEM((NW * 16,), i32),
            pltpu.VMEM((BLK,), i32),
            pltpu.VMEM((BLK,), i32),
            pltpu.VMEM((BLK,), f32),
            pltpu.VMEM((RPT,), f32),
            pltpu.VMEM((RPT + 16,), f32),
            pltpu.VMEM((SUB, D), f32),
            pltpu.SemaphoreType.DMA,
            pltpu.SemaphoreType.DMA,
            pltpu.SemaphoreType.DMA,
            pltpu.SemaphoreType.DMA,
        ],
    )
    def k(dl_h, cnt_h, x_h, s_h, m0_h, degspm, cbuf, ib0, ib1, ones, dbuf,
          sbufv, xb, d0, d1, s0, s1):
        idxb = (ib0, ib1)
        dsem = (d0, d1)
        ssem = (s0, s1)
        c = lax.axis_index("core")
        t = lax.axis_index("subcore")

        z16 = jnp.zeros((16,), f32)
        for j in range(RPT // 16):
            dbuf[pl.ds(j * 16, 16)] = z16
        o16 = jnp.ones((16,), f32)
        for j in range(BLK // 16):
            ones[pl.ds(j * 16, 16)] = o16
        pltpu.sync_copy(dbuf, degspm.at[pl.ds(t * RPT, RPT)])
        pltpu.sync_copy(cnt_h.at[pl.ds(c * NW * 16, NW * 16)], cbuf)
        plsc.subcore_barrier()

        hbase = (c * NW + 2 * t) * CAP
        cv0 = cbuf[pl.ds((2 * t) * 16, 16)][0]
        cv1 = cbuf[pl.ds((2 * t + 1) * 16, 16)][0]
        nb0 = (cv0 + BLK - 1) // BLK
        nbt = nb0 + (cv1 + BLK - 1) // BLK

        def _wait_dscatter(q):
            pltpu.make_async_copy(ones, degspm.at[idxb[q]], ssem[q]).wait()

        def _wait_didx(q):
            pltpu.make_async_copy(dl_h.at[pl.ds(0, BLK)], idxb[q],
                                  dsem[q]).wait()

        def dgroup(g, _):
            for u_ in range(2):
                b = g * 2 + u_

                @pl.when(b < nbt)
                def _(b=b, u_=u_):
                    @pl.when(b >= 2)
                    def _():
                        _wait_dscatter(u_)

                    voff = jnp.where(b < nb0, b * BLK,
                                     CAP + (b - nb0) * BLK)
                    pltpu.async_copy(dl_h.at[pl.ds(hbase + voff, BLK)],
                                     idxb[u_], dsem[u_])

                    @pl.when(b >= 1)
                    def _():
                        q = (u_ + 1) % 2
                        _wait_didx(q)
                        pltpu.async_copy(ones, degspm.at[idxb[q]],
                                         ssem[q], add=True)
            return 0

        lax.fori_loop(0, (nbt + 1) // 2, dgroup, 0)
        for q in range(2):
            @pl.when(jnp.logical_and(nbt >= 1, (nbt - 1) % 2 == q))
            def _(q=q):
                _wait_didx(q)
                pltpu.async_copy(ones, degspm.at[idxb[q]], ssem[q], add=True)
        for q in range(2):
            @pl.when(nbt > q)
            def _(q=q):
                _wait_dscatter(q)
        plsc.subcore_barrier()

        pltpu.sync_copy(degspm.at[pl.ds(t * RPT, RPT)], dbuf)
        for j in range(RPT // 16):
            dv = dbuf[pl.ds(j * 16, 16)]
            sbufv[pl.ds(j * 16, 16)] = _rsqrt16(jnp.maximum(dv, 1.0))
        pltpu.sync_copy(sbufv.at[pl.ds(0, RPT)], s_h.at[pl.ds(c * UROWS + t * RPT, RPT)])

        rc = jnp.minimum(RPT, HALF - t * RPT)
        nblk = rc // SUB

        def mblk(b, _):
            g0 = c * HALF + t * RPT + b * SUB
            pltpu.sync_copy(x_h.at[pl.ds(g0, SUB)], xb)

            def row(i, _):
                sr = sbufv[pl.ds(b * SUB + i, 16)][0]
                for j in range(D // 16):
                    sl = pl.ds(j * 16, 16)
                    xb[i, sl] = xb[i, sl] * sr
                return 0

            lax.fori_loop(0, SUB, row, 0)
            pltpu.sync_copy(xb, m0_h.at[pl.ds(g0, SUB)])
            return 0

        lax.fori_loop(0, nblk, mblk, 0)

    return k(dl, cnt, x)


@jax.jit
def _layer_sc(m, sl, dl, cnt):
    @functools.partial(
        pl.kernel,
        out_type=jax.ShapeDtypeStruct((NC * UROWS, D), f32),
        mesh=_mesh,
        compiler_params=_cparams,
        scratch_types=[
            pltpu.VMEM_SHARED((UROWS, D), f32),
            pltpu.VMEM((SUB, D), f32),
            pltpu.VMEM((NW * 16,), i32),
            pltpu.VMEM((2 * CAP,), i32),
            pltpu.VMEM((BLK,), i32),
            pltpu.VMEM((BLK,), i32),
            pltpu.VMEM((BLK,), i32),
            pltpu.VMEM((BLK, D), f32),
            pltpu.VMEM((BLK, D), f32),
            pltpu.VMEM((BLK, D), f32),
            pltpu.SemaphoreType.DMA,
            pltpu.SemaphoreType.DMA,
            pltpu.SemaphoreType.DMA,
            pltpu.SemaphoreType.DMA,
            pltpu.SemaphoreType.DMA,
            pltpu.SemaphoreType.DMA,
            pltpu.SemaphoreType.DMA,
            pltpu.SemaphoreType.DMA,
            pltpu.SemaphoreType.DMA,
            pltpu.SemaphoreType.DMA,
        ],
    )
    def k(m_h, sl_h, dl_h, cnt_h, u_h, uspm, zb, cbuf, srcl,
          id0, id1, id2, rw0, rw1, rw2,
          d0, d1, d2, g0, g1, g2, s0, s1, s2, pf):
        idst = (id0, id1, id2)
        rows = (rw0, rw1, rw2)
        dsem = (d0, d1, d2)
        gsem = (g0, g1, g2)
        ssem = (s0, s1, s2)
        c = lax.axis_index("core")
        t = lax.axis_index("subcore")
        hbase = (c * NW + 2 * t) * CAP

        pfds = [
            pltpu.async_copy(sl_h.at[pl.ds(hbase, CAP)],
                             srcl.at[pl.ds(0, CAP)], pf),
            pltpu.async_copy(sl_h.at[pl.ds(hbase + CAP, CAP)],
                             srcl.at[pl.ds(CAP, CAP)], pf),
        ]

        z16 = jnp.zeros((16,), f32)
        for i in range(SUB):
            for j in range(D // 16):
                zb[i, pl.ds(j * 16, 16)] = z16
        for b in range(RPT // SUB):
            pltpu.sync_copy(zb, uspm.at[pl.ds(t * RPT + b * SUB, SUB)])
        pltpu.sync_copy(cnt_h.at[pl.ds(c * NW * 16, NW * 16)], cbuf)
        for dsc in pfds:
            dsc.wait()
        plsc.subcore_barrier()

        cv0 = cbuf[pl.ds((2 * t) * 16, 16)][0]
        cv1 = cbuf[pl.ds((2 * t + 1) * 16, 16)][0]
        nb0 = (cv0 + BLK - 1) // BLK
        nbt = nb0 + (cv1 + BLK - 1) // BLK

        def voff_of(b):
            return jnp.where(b < nb0, b * BLK, CAP + (b - nb0) * BLK)

        def _wait_scatter(q):
            pltpu.make_async_copy(rows[q], uspm.at[idst[q]], ssem[q]).wait()

        def _wait_gather(q):
            pltpu.make_async_copy(
                m_h.at[srcl.at[pl.ds(0, BLK)]], rows[q], gsem[q]).wait()

        def _wait_idx(q):
            pltpu.make_async_copy(dl_h.at[pl.ds(0, BLK)], idst[q],
                                  dsem[q]).wait()

        def _issue(b, p):
            voff = voff_of(b)
            pltpu.async_copy(dl_h.at[pl.ds(hbase + voff, BLK)],
                             idst[p], dsem[p])
            pltpu.async_copy(m_h.at[srcl.at[pl.ds(voff, BLK)]],
                             rows[p], gsem[p])

        def _fire_scatter(q):
            pltpu.async_copy(rows[q], uspm.at[idst[q]], ssem[q], add=True)

        def group(g, _):
            for u_ in range(3):
                b = g * 3 + u_

                @pl.when(b < nbt)
                def _(b=b, u_=u_):
                    @pl.when(b >= 3)
                    def _():
                        _wait_scatter(u_)

                    _issue(b, u_)

                    @pl.when(b >= 2)
                    def _():
                        q = (u_ + 1) % 3
                        _wait_idx(q)
                        _wait_gather(q)
                        _fire_scatter(q)
            return 0

        lax.fori_loop(0, (nbt + 2) // 3, group, 0)
        for bb_off in (2, 1):
            for q in range(3):
                @pl.when(jnp.logical_and(nbt >= bb_off,
                                         (nbt - bb_off) % 3 == q))
                def _(q=q):
                    _wait_idx(q)
                    _wait_gather(q)
                    _fire_scatter(q)
        for q in range(3):
            @pl.when(nbt > q)
            def _(q=q):
                _wait_scatter(q)
        plsc.subcore_barrier()

        pltpu.sync_copy(uspm.at[pl.ds(t * RPT, RPT)],
                        u_h.at[pl.ds(c * UROWS + t * RPT, RPT)])

    return k(m, sl, dl, cnt)


def _make_combine(scale, write_m):
    out_type = [jax.ShapeDtypeStruct((N_NODES, D), f32)]
    if write_m:
        out_type.append(jax.ShapeDtypeStruct((N_NODES, D), f32))

    CR = 40

    def body(u_ref, s_ref, acc_ref, *outs):
        acc_out = outs[0]
        sv = s_ref[...]
        out = sv * u_ref[...]
        acc_out[...] = (acc_ref[...] + out) * scale
        if write_m:
            outs[1][...] = sv * out

    def umap(i):
        return (i + 3 * ((CR * i) // HALF), 0)

    @jax.jit
    def run(u, s, acc_in):
        return pl.pallas_call(
            body,
            grid=(N_NODES // CR,),
            in_specs=[
                pl.BlockSpec((CR, D), umap),
                pl.BlockSpec((CR, 1), umap),
                pl.BlockSpec((CR, D), lambda i: (i, 0)),
            ],
            out_specs=[pl.BlockSpec((CR, D), lambda i: (i, 0))] * len(out_type),
            out_shape=tuple(out_type),
        )(u, s.reshape(NC * UROWS, 1), acc_in)

    return run


_combine_mid = _make_combine(1.0, True)
_combine_last = _make_combine(0.25, False)


@jax.jit
def _user_gather(acc, user_idx):
    per_w = 4096 // NW

    @functools.partial(
        pl.kernel,
        out_type=jax.ShapeDtypeStruct((4096, D), f32),
        mesh=_mesh,
        compiler_params=_cparams,
        scratch_types=[
            pltpu.VMEM((per_w,), i32),
            pltpu.VMEM((per_w, D), f32),
        ],
    )
    def k(acc_h, uid_h, out_h, idxb, rowsb):
        c = lax.axis_index("core")
        t = lax.axis_index("subcore")
        w = c * NS + t
        pltpu.sync_copy(uid_h.at[pl.ds(w * per_w, per_w)], idxb)
        pltpu.sync_copy(acc_h.at[idxb], rowsb)
        pltpu.sync_copy(rowsb, out_h.at[pl.ds(w * per_w, per_w)])

    return k(acc, user_idx)


def kernel(x, user_idx, edge_index):
    src = edge_index[0]
    dst = edge_index[1]
    sl, dl, cnt = _p1(src, dst)
    s, m0 = _p2(dl, cnt, x)
    u1 = _layer_sc(m0, sl, dl, cnt)
    acc1, m1 = _combine_mid(u1, s, x)
    u2 = _layer_sc(m1, sl, dl, cnt)
    acc2, m2 = _combine_mid(u2, s, acc1)
    u3 = _layer_sc(m2, sl, dl, cnt)
    (acc3,) = _combine_last(u3, s, acc2)
    user_embedding = _user_gather(acc3, user_idx)
    item_embedding = acc3[N_USERS:]
    return (user_embedding, item_embedding)


# speedup vs baseline: 1.6608x; 1.6608x over previous
import dataclasses
import functools

import jax
import jax.numpy as jnp
from jax import lax
from jax.experimental import pallas as pl
from jax.experimental.pallas import tpu as pltpu
from jax.experimental.pallas import tpu_sc as plsc

N_USERS = 3000
N_NODES = 10000
D = 128
N_EDGES = 320000
HALF = 5000

NC = 2
NS = 16
NW = NC * NS
CHUNK = N_EDGES // NW
CAP = 10240
NG = 4
TRASH = 5000
RPT = 320
LAST = HALF - (NS - 1) * RPT
UROWS = NS * RPT
BLK = 128
LBLK = 64
SUB = 40

_mesh = plsc.VectorSubcoreMesh(core_axis_name="core", subcore_axis_name="subcore")

_cparams = pltpu.CompilerParams()
if "needs_layout_passes" in pltpu.CompilerParams.__dataclass_fields__:
    _cparams = dataclasses.replace(_cparams, needs_layout_passes=False)

f32 = jnp.float32
i32 = jnp.int32


def _rsqrt16(x):
    i = plsc.bitcast(x, i32)
    i = jnp.int32(0x5F3759DF) - (i >> 1)
    y = plsc.bitcast(i, f32)
    for _ in range(4):
        y = y * (1.5 - 0.5 * x * y * y)
    return y


@jax.jit
def _p1(src, dst):
    @functools.partial(
        pl.kernel,
        out_type=(
            jax.ShapeDtypeStruct((NG * NW * CAP,), i32),
            jax.ShapeDtypeStruct((NG * NW * CAP,), i32),
            jax.ShapeDtypeStruct((NG * NW * 16,), i32),
        ),
        mesh=_mesh,
        compiler_params=_cparams,
        scratch_types=[
            pltpu.VMEM((CHUNK,), i32),
            pltpu.VMEM((CHUNK,), i32),
            pltpu.VMEM((CAP,), i32),
            pltpu.VMEM((CAP,), i32),
            pltpu.VMEM((CAP,), i32),
            pltpu.VMEM((CAP,), i32),
            pltpu.VMEM((CAP,), i32),
            pltpu.VMEM((CAP,), i32),
            pltpu.VMEM((CAP,), i32),
            pltpu.VMEM((CAP,), i32),
            pltpu.VMEM((16,), i32),
        ],
    )
    def k(src_h, dst_h, sl_h, dl_h, cnt_h, sbuf, dbuf,
          l0s, l0d, l1s, l1d, l2s, l2d, l3s, l3d, cbuf):
        c = lax.axis_index("core")
        t = lax.axis_index("subcore")
        w = c * NS + t
        pltpu.sync_copy(src_h.at[pl.ds(w * CHUNK, CHUNK)], sbuf)
        pltpu.sync_copy(dst_h.at[pl.ds(w * CHUNK, CHUNK)], dbuf)

        ls = (l0s, l1s, l2s, l3s)
        ld = (l0d, l1d, l2d, l3d)

        def step(i, ns):
            n0, n1, n2, n3 = ns
            sv = sbuf[pl.ds(i * 16, 16)]
            dv = dbuf[pl.ds(i * 16, 16)]
            mdst0 = dv < HALF
            msrc0 = sv < HALF
            m0 = jnp.logical_and(mdst0, msrc0)
            m1 = jnp.logical_and(mdst0, jnp.logical_not(msrc0))
            m2 = jnp.logical_and(jnp.logical_not(mdst0),
                                 jnp.logical_not(msrc0))
            m3 = jnp.logical_and(jnp.logical_not(mdst0), msrc0)
            plsc.store_compressed(l0s.at[pl.ds(n0, 16)], sv, mask=m0)
            plsc.store_compressed(l0d.at[pl.ds(n0, 16)], dv, mask=m0)
            plsc.store_compressed(l1s.at[pl.ds(n1, 16)], sv, mask=m1)
            plsc.store_compressed(l1d.at[pl.ds(n1, 16)], dv, mask=m1)
            plsc.store_compressed(l2s.at[pl.ds(n2, 16)], sv - HALF, mask=m2)
            plsc.store_compressed(l2d.at[pl.ds(n2, 16)], dv - HALF, mask=m2)
            plsc.store_compressed(l3s.at[pl.ds(n3, 16)], sv, mask=m3)
            plsc.store_compressed(l3d.at[pl.ds(n3, 16)], dv - HALF, mask=m3)
            n0 = n0 + plsc.all_reduce_population_count(m0)[0]
            n1 = n1 + plsc.all_reduce_population_count(m1)[0]
            n2 = n2 + plsc.all_reduce_population_count(m2)[0]
            n3 = n3 + plsc.all_reduce_population_count(m3)[0]
            return n0, n1, n2, n3

        ns = lax.fori_loop(0, CHUNK // 16, step,
                           (i32(0), i32(0), i32(0), i32(0)))

        zeros16 = jnp.zeros((16,), i32)
        trash16 = jnp.full((16,), TRASH, i32)
        for g in range(NG):
            for j in range(BLK // 16):
                ls[g][pl.ds(ns[g] + j * 16, 16)] = zeros16
                ld[g][pl.ds(ns[g] + j * 16, 16)] = trash16

        for g in range(NG):
            pltpu.sync_copy(ls[g], sl_h.at[pl.ds((g * NW + w) * CAP, CAP)])
            pltpu.sync_copy(ld[g], dl_h.at[pl.ds((g * NW + w) * CAP, CAP)])
            cbuf[pl.ds(0, 16)] = jnp.full((16,), 1, i32) * ns[g]
            pltpu.sync_copy(cbuf, cnt_h.at[pl.ds((g * NW + w) * 16, 16)])

    return k(src, dst)


def _block_plan(cbuf, c, t, blk):
    gl, gr = 2 * c, 2 * c + 1

    def nb(g, w):
        cv = cbuf[pl.ds((g * NW + w) * 16, 16)][0]
        return (cv + blk - 1) // blk

    nb0 = nb(gl, 2 * t)
    cumL = nb0 + nb(gl, 2 * t + 1)
    cum2 = cumL + nb(gr, 2 * t)
    nbt = cum2 + nb(gr, 2 * t + 1)

    def voff_of(b):
        return jnp.where(
            b < nb0, (gl * NW + 2 * t) * CAP + b * blk,
            jnp.where(
                b < cumL, (gl * NW + 2 * t + 1) * CAP + (b - nb0) * blk,
                jnp.where(
                    b < cum2, (gr * NW + 2 * t) * CAP + (b - cumL) * blk,
                    (gr * NW + 2 * t + 1) * CAP + (b - cum2) * blk)))

    return nb0, cumL, cum2, nbt, voff_of


@jax.jit
def _p2(dl, cnt, x):
    @functools.partial(
        pl.kernel,
        out_type=(
            jax.ShapeDtypeStruct((NC * UROWS,), f32),
            jax.ShapeDtypeStruct((N_NODES, D), f32),
        ),
        mesh=_mesh,
        compiler_params=_cparams,
        scratch_types=[
            pltpu.VMEM_SHARED((UROWS,), f32),
            pltpu.VMEM((NG * NW * 16,), i32),
            pltpu.VMEM((BLK,), i32),
            pltpu.VMEM((BLK,), i32),
            pltpu.VMEM((BLK,), f32),
            pltpu.VMEM((RPT,), f32),
            pltpu.VMEM((RPT + 16,), f32),
            pltpu.VMEM((SUB, D), f32),
            pltpu.SemaphoreType.DMA,
            pltpu.SemaphoreType.DMA,
            pltpu.SemaphoreType.DMA,
            pltpu.SemaphoreType.DMA,
        ],
    )
    def k(dl_h, cnt_h, x_h, s_h, m0_h, degspm, cbuf, ib0, ib1, ones, dbuf,
          sbufv, xb, d0, d1, s0, s1):
        idxb = (ib0, ib1)
        dsem = (d0, d1)
        ssem = (s0, s1)
        c = lax.axis_index("core")
        t = lax.axis_index("subcore")

        z16 = jnp.zeros((16,), f32)
        for j in range(RPT // 16):
            dbuf[pl.ds(j * 16, 16)] = z16
        o16 = jnp.ones((16,), f32)
        for j in range(BLK // 16):
            ones[pl.ds(j * 16, 16)] = o16
        pltpu.sync_copy(dbuf, degspm.at[pl.ds(t * RPT, RPT)])
        pltpu.sync_copy(cnt_h, cbuf)
        plsc.subcore_barrier()

        _, _, _, nbt, voff_of = _block_plan(cbuf, c, t, BLK)

        def _wait_dscatter(q):
            pltpu.make_async_copy(ones, degspm.at[idxb[q]], ssem[q]).wait()

        def _wait_didx(q):
            pltpu.make_async_copy(dl_h.at[pl.ds(0, BLK)], idxb[q],
                                  dsem[q]).wait()

        def dgroup(g, _):
            for u_ in range(2):
                b = g * 2 + u_

                @pl.when(b < nbt)
                def _(b=b, u_=u_):
                    @pl.when(b >= 2)
                    def _():
                        _wait_dscatter(u_)

                    pltpu.async_copy(dl_h.at[pl.ds(voff_of(b), BLK)],
                                     idxb[u_], dsem[u_])

                    @pl.when(b >= 1)
                    def _():
                        q = (u_ + 1) % 2
                        _wait_didx(q)
                        pltpu.async_copy(ones, degspm.at[idxb[q]],
                                         ssem[q], add=True)
            return 0

        lax.fori_loop(0, (nbt + 1) // 2, dgroup, 0)
        for q in range(2):
            @pl.when(jnp.logical_and(nbt >= 1, (nbt - 1) % 2 == q))
            def _(q=q):
                _wait_didx(q)
                pltpu.async_copy(ones, degspm.at[idxb[q]], ssem[q], add=True)
        for q in range(2):
            @pl.when(nbt > q)
            def _(q=q):
                _wait_dscatter(q)
        plsc.subcore_barrier()

        pltpu.sync_copy(degspm.at[pl.ds(t * RPT, RPT)], dbuf)
        for j in range(RPT // 16):
            dv = dbuf[pl.ds(j * 16, 16)]
            sbufv[pl.ds(j * 16, 16)] = _rsqrt16(jnp.maximum(dv, 1.0))
        pltpu.sync_copy(sbufv.at[pl.ds(0, RPT)],
                        s_h.at[pl.ds(c * UROWS + t * RPT, RPT)])

        rc = jnp.minimum(RPT, HALF - t * RPT)
        nblk = rc // SUB

        def mblk(b, _):
            g0 = c * HALF + t * RPT + b * SUB
            pltpu.sync_copy(x_h.at[pl.ds(g0, SUB)], xb)

            def row(i, _):
                sr = sbufv[pl.ds(b * SUB + i, 16)][0]
                for j in range(D // 16):
                    sl = pl.ds(j * 16, 16)
                    xb[i, sl] = xb[i, sl] * sr
                return 0

            lax.fori_loop(0, SUB, row, 0)
            pltpu.sync_copy(xb, m0_h.at[pl.ds(g0, SUB)])
            return 0

        lax.fori_loop(0, nblk, mblk, 0)

    return k(dl, cnt, x)


def _make_layer(scale, write_m, gather_users=False):
    out_type = [jax.ShapeDtypeStruct((N_NODES, D), f32)]
    if write_m:
        out_type.append(jax.ShapeDtypeStruct((N_NODES, D), f32))
    if gather_users:
        out_type.append(jax.ShapeDtypeStruct((4096, D), f32))

    @jax.jit
    def run(m, sl, dl, cnt, s, acc_in, uid=None):
        args = [m, sl, dl, cnt, s, acc_in]
        if gather_users:
            args.append(uid)

        @functools.partial(
            pl.kernel,
            out_type=tuple(out_type),
            mesh=_mesh,
            compiler_params=_cparams,
            scratch_types=[
                pltpu.VMEM_SHARED((UROWS, D), f32),
                pltpu.VMEM_SHARED((UROWS, D), f32),
                pltpu.VMEM((SUB, D), f32),
                pltpu.VMEM((NG * NW * 16,), i32),
                pltpu.VMEM((LBLK,), i32),
                pltpu.VMEM((LBLK,), i32),
                pltpu.VMEM((LBLK,), i32),
                pltpu.VMEM((LBLK,), i32),
                pltpu.VMEM((LBLK,), i32),
                pltpu.VMEM((LBLK,), i32),
                pltpu.VMEM((LBLK, D), f32),
                pltpu.VMEM((LBLK, D), f32),
                pltpu.VMEM((LBLK, D), f32),
                pltpu.VMEM((SUB, D), f32),
                pltpu.VMEM((RPT + 16,), f32),
                pltpu.SemaphoreType.DMA,
                pltpu.SemaphoreType.DMA,
                pltpu.SemaphoreType.DMA,
                pltpu.SemaphoreType.DMA,
                pltpu.SemaphoreType.DMA,
                pltpu.SemaphoreType.DMA,
                pltpu.SemaphoreType.DMA,
                pltpu.SemaphoreType.DMA,
                pltpu.SemaphoreType.DMA,
                pltpu.SemaphoreType.DMA,
                pltpu.SemaphoreType.DMA,
                pltpu.SemaphoreType.DMA,
                pltpu.SemaphoreType.DMA,
            ],
        )
        def k(*refs):
            n_in = 7 if gather_users else 6
            m_h, sl_h, dl_h, cnt_h, s_h, acc_in_h = refs[:6]
            uid_h = refs[6] if gather_users else None
            pos = n_in
            acc_out_h = refs[pos]; pos += 1
            m_out_h = None
            ue_h = None
            if write_m:
                m_out_h = refs[pos]; pos += 1
            if gather_users:
                ue_h = refs[pos]; pos += 1
            (uspm, mspm, zb, cbuf, is0, is1, is2, id0, id1, id2,
             rw0, rw1, rw2, ab, sb,
             i0, i1, i2, d0, d1, d2, g0, g1, g2, s0, s1, s2, pf) = refs[pos:]
            isrc = (is0, is1, is2)
            idst = (id0, id1, id2)
            rows = (rw0, rw1, rw2)
            isem = (i0, i1, i2)
            dsem = (d0, d1, d2)
            gsem = (g0, g1, g2)
            ssem = (s0, s1, s2)
            c = lax.axis_index("core")
            t = lax.axis_index("subcore")

            @pl.when(t < NS - 1)
            def _():
                pltpu.async_copy(m_h.at[pl.ds(c * HALF + t * RPT, RPT)],
                                 mspm.at[pl.ds(t * RPT, RPT)], pf)

            @pl.when(t == NS - 1)
            def _():
                pltpu.async_copy(
                    m_h.at[pl.ds(c * HALF + (NS - 1) * RPT, LAST)],
                    mspm.at[pl.ds((NS - 1) * RPT, LAST)], pf)

            z16 = jnp.zeros((16,), f32)
            for i in range(SUB):
                for j in range(D // 16):
                    zb[i, pl.ds(j * 16, 16)] = z16
            for b in range(RPT // SUB):
                pltpu.sync_copy(zb, uspm.at[pl.ds(t * RPT + b * SUB, SUB)])
            pltpu.sync_copy(cnt_h, cbuf)
            pltpu.sync_copy(s_h.at[pl.ds(c * UROWS + t * RPT, RPT)],
                            sb.at[pl.ds(0, RPT)])

            @pl.when(t < NS - 1)
            def _():
                pltpu.make_async_copy(m_h.at[pl.ds(0, RPT)],
                                      mspm.at[pl.ds(t * RPT, RPT)],
                                      pf).wait()

            @pl.when(t == NS - 1)
            def _():
                pltpu.make_async_copy(
                    m_h.at[pl.ds(0, LAST)],
                    mspm.at[pl.ds((NS - 1) * RPT, LAST)], pf).wait()
            plsc.subcore_barrier()

            _, cumL, _, nbt, voff_of = _block_plan(cbuf, c, t, LBLK)

            def _wait_scatter(q):
                pltpu.make_async_copy(rows[q], uspm.at[idst[q]],
                                      ssem[q]).wait()

            def group(g_, _):
                for u_ in range(3):
                    b = g_ * 3 + u_

                    @pl.when(b <= nbt + 1)
                    def _(b=b, u_=u_):
                        @pl.when(jnp.logical_and(b >= 1, b - 1 < nbt))
                        def _():
                            q = (u_ + 2) % 3
                            pltpu.make_async_copy(
                                sl_h.at[pl.ds(0, LBLK)], isrc[q],
                                isem[q]).wait()

                            @pl.when(b - 1 < cumL)
                            def _():
                                pltpu.async_copy(mspm.at[isrc[q]], rows[q],
                                                 gsem[q])

                            @pl.when(b - 1 >= cumL)
                            def _():
                                pltpu.async_copy(m_h.at[isrc[q]], rows[q],
                                                 gsem[q])

                        @pl.when(jnp.logical_and(b >= 2, b - 2 < nbt))
                        def _():
                            r = (u_ + 1) % 3
                            pltpu.make_async_copy(
                                dl_h.at[pl.ds(0, LBLK)], idst[r],
                                dsem[r]).wait()
                            pltpu.make_async_copy(
                                m_h.at[isrc[r]], rows[r], gsem[r]).wait()
                            pltpu.async_copy(rows[r], uspm.at[idst[r]],
                                             ssem[r], add=True)

                        @pl.when(b < nbt)
                        def _():
                            @pl.when(b >= 3)
                            def _():
                                _wait_scatter(u_)

                            voff = voff_of(b)
                            pltpu.async_copy(sl_h.at[pl.ds(voff, LBLK)],
                                             isrc[u_], isem[u_])
                            pltpu.async_copy(dl_h.at[pl.ds(voff, LBLK)],
                                             idst[u_], dsem[u_])
                return 0

            lax.fori_loop(0, (nbt + 4) // 3, group, 0)
            for q in range(3):
                @pl.when(nbt > q)
                def _(q=q):
                    _wait_scatter(q)
            plsc.subcore_barrier()

            rc = jnp.minimum(RPT, HALF - t * RPT)
            nblk = rc // SUB

            def mblk(b, _):
                l0 = t * RPT + b * SUB
                g0_ = c * HALF + l0
                pltpu.sync_copy(uspm.at[pl.ds(l0, SUB)], zb)
                pltpu.sync_copy(acc_in_h.at[pl.ds(g0_, SUB)], ab)

                def row(i, _):
                    sr = sb[pl.ds(b * SUB + i, 16)][0]
                    for j in range(D // 16):
                        sl_ = pl.ds(j * 16, 16)
                        out16 = zb[i, sl_] * sr
                        ab[i, sl_] = (ab[i, sl_] + out16) * scale
                        if write_m:
                            zb[i, sl_] = out16 * sr
                    return 0

                lax.fori_loop(0, SUB, row, 0)
                pltpu.sync_copy(ab, acc_out_h.at[pl.ds(g0_, SUB)])
                if write_m:
                    pltpu.sync_copy(zb, m_out_h.at[pl.ds(g0_, SUB)])
                return 0

            lax.fori_loop(0, nblk, mblk, 0)

            if gather_users:
                plsc.subcore_barrier()

                @pl.when(c == 0)
                def _():
                    per_t = 4096 // NS
                    for g_ in range(per_t // LBLK):
                        u0 = t * per_t + g_ * LBLK
                        pltpu.sync_copy(uid_h.at[pl.ds(u0, LBLK)], is0)
                        pltpu.sync_copy(acc_out_h.at[is0], rw0)
                        pltpu.sync_copy(rw0, ue_h.at[pl.ds(u0, LBLK)])

        return k(*args)

    return run


_layer_mid = _make_layer(1.0, True)
_layer_last = _make_layer(0.25, False, gather_users=True)


def kernel(x, user_idx, edge_index):
    src = edge_index[0]
    dst = edge_index[1]
    sl, dl, cnt = _p1(src, dst)
    s, m0 = _p2(dl, cnt, x)
    acc1, m1 = _layer_mid(m0, sl, dl, cnt, s, x)
    acc2, m2 = _layer_mid(m1, sl, dl, cnt, s, acc1)
    acc3, user_embedding = _layer_last(m2, sl, dl, cnt, s, acc2, user_idx)
    item_embedding = acc3[N_USERS:]
    return (user_embedding, item_embedding)
